# baseline (speedup 1.0000x reference)
import jax
import jax.numpy as jnp
from jax.experimental import pallas as pl
from jax.experimental.pallas import tpu as pltpu

N_CLASS = 5
NS = 5
D = 768
HW = 196
K1 = 20
K2 = 10
M = NS * HW
MP = 1024
ML = MP // 4
BQ = 75
QB = 5
NG = BQ // QB
W = QB * HW


def _body(s_ref, q_ref, o_ref, sim_ref, t1_ref):
    S = s_ref[0]
    Q = q_ref[0]
    rs = jax.lax.rsqrt(jnp.sum(S * S, axis=1))
    rq = jax.lax.rsqrt(jnp.sum(Q * Q, axis=0))
    raw = jax.lax.dot_general(
        S.astype(jnp.bfloat16), Q.astype(jnp.bfloat16),
        (((1,), (0,)), ((), ())),
        preferred_element_type=jnp.float32)
    sim = raw * rs[:, None]
    row_iota = jax.lax.broadcasted_iota(jnp.int32, (MP, W), 0)
    simb = jnp.where(row_iota < M, sim, -jnp.inf).astype(jnp.bfloat16)

    a = simb[0 * ML:1 * ML]
    b = simb[1 * ML:2 * ML]
    c = simb[2 * ML:3 * ML]
    d = simb[3 * ML:4 * ML]
    a, b = jnp.maximum(a, b), jnp.minimum(a, b)
    c, d = jnp.maximum(c, d), jnp.minimum(c, d)
    a, c = jnp.maximum(a, c), jnp.minimum(a, c)
    b, d = jnp.maximum(b, d), jnp.minimum(b, d)
    b, c = jnp.maximum(b, c), jnp.minimum(b, c)
    sim_ref[0 * ML:1 * ML] = a
    sim_ref[1 * ML:2 * ML] = b
    sim_ref[2 * ML:3 * ML] = c
    sim_ref[3 * ML:4 * ML] = d

    vs, bs = [], []
    before = jnp.zeros((W,), jnp.float32)
    m = jnp.max(a, axis=0)
    one = jnp.ones((), jnp.bfloat16)
    zero = jnp.zeros((), jnp.bfloat16)
    ones_row = jnp.ones((1, ML), jnp.bfloat16)
    for i in range(K1):
        vs.append(m.astype(jnp.float32) * rq)
        bs.append(before)
        if i < K1 - 1:
            t0 = sim_ref[0 * ML:1 * ML]
            t1 = sim_ref[1 * ML:2 * ML]
            t2 = sim_ref[2 * ML:3 * ML]
            t3 = sim_ref[3 * ML:4 * ML]
            eq = t0 == m[None, :]
            eqb = jnp.where(eq, one, zero)
            cnt = jax.lax.dot_general(
                ones_row, eqb, (((1,), (0,)), ((), ())),
                preferred_element_type=jnp.float32)
            before = before + cnt[0]
            nt0 = jnp.where(eq, t1, t0)
            sim_ref[0 * ML:1 * ML] = nt0
            sim_ref[1 * ML:2 * ML] = jnp.where(eq, t2, t1)
            sim_ref[2 * ML:3 * ML] = jnp.where(eq, t3, t2)
            sim_ref[3 * ML:4 * ML] = jnp.where(eq, -jnp.inf, t3)
            m = jnp.max(nt0, axis=0)
    j_iota = jax.lax.broadcasted_iota(jnp.int32, (K1, W), 0).astype(jnp.float32)
    t1 = jnp.full((K1, W), jnp.inf)
    for v, bc in zip(vs, bs):
        t1 = jnp.minimum(t1, jnp.where(bc[None, :] <= j_iota, v[None, :], jnp.inf))
    t1_ref[...] = t1

    i_iota = jax.lax.broadcasted_iota(jnp.int32, (K1, K2), 1)
    for q in range(QB):
        cur = t1_ref[:, q * HW:(q + 1) * HW]
        v2s, b2s = [], []
        before2 = jnp.zeros((K1,), jnp.int32)
        m2 = jnp.max(cur, axis=1)
        for i in range(K2):
            v2s.append(m2)
            b2s.append(before2)
            if i < K2 - 1:
                eq2 = cur == m2[:, None]
                before2 = before2 + jnp.sum(eq2.astype(jnp.int32), axis=1)
                cur = jnp.where(eq2, -jnp.inf, cur)
                m2 = jnp.max(cur, axis=1)
        out = jnp.full((K1, K2), jnp.inf)
        for v, bc in zip(v2s, b2s):
            out = jnp.minimum(out, jnp.where(bc[:, None] <= i_iota, v[:, None], jnp.inf))
        o_ref[q] = out


def kernel(support, query, task_index, special_list, mode, k, k2):
    s5 = support.reshape(N_CLASS, NS, D, HW).transpose(0, 1, 3, 2)
    s5 = s5.reshape(N_CLASS, M, D)
    s5 = jnp.pad(s5, ((0, 0), (0, MP - M), (0, 0)))
    q5 = query.reshape(NG, QB, D, HW).transpose(0, 2, 1, 3).reshape(NG, D, W)

    out = pl.pallas_call(
        _body,
        grid=(N_CLASS, NG),
        in_specs=[
            pl.BlockSpec((1, MP, D), lambda c, g: (c, 0, 0)),
            pl.BlockSpec((1, D, W), lambda c, g: (g, 0, 0)),
        ],
        out_specs=pl.BlockSpec((QB, K1, K2), lambda c, g: (c * NG + g, 0, 0)),
        out_shape=jax.ShapeDtypeStruct((N_CLASS * BQ, K1, K2), jnp.float32),
        scratch_shapes=[pltpu.VMEM((MP, W), jnp.bfloat16),
                        pltpu.VMEM((K1, W), jnp.float32)],
    )(s5, q5)

    zero = (jnp.asarray(k) - K1) + (jnp.asarray(k2) - K2)
    return out.reshape(N_CLASS * BQ, 1, K1, K2) + zero.astype(out.dtype)


# speedup vs baseline: 1.0125x; 1.0125x over previous
import jax
import jax.numpy as jnp
from jax.experimental import pallas as pl
from jax.experimental.pallas import tpu as pltpu

N_CLASS = 5
NS = 5
D = 768
HW = 196
K1 = 20
K2 = 10
M = NS * HW
MP = 1024
ML = MP // 4
BQ = 75
QB = 5
NG = BQ // QB
W = QB * HW


def _body(s_ref, q_ref, o_ref, sim_ref, t1_ref):
    S = s_ref[0]
    Q = q_ref[0]
    rs = jax.lax.rsqrt(jnp.sum(S * S, axis=1, dtype=jnp.float32))
    rq = jax.lax.rsqrt(jnp.sum(Q * Q, axis=0, dtype=jnp.float32))
    raw = jax.lax.dot_general(
        S, Q, (((1,), (0,)), ((), ())),
        preferred_element_type=jnp.float32)
    sim = raw * rs[:, None]
    row_iota = jax.lax.broadcasted_iota(jnp.int32, (MP, W), 0)
    simb = jnp.where(row_iota < M, sim, -jnp.inf).astype(jnp.bfloat16)

    a = simb[0 * ML:1 * ML]
    b = simb[1 * ML:2 * ML]
    c = simb[2 * ML:3 * ML]
    d = simb[3 * ML:4 * ML]
    a, b = jnp.maximum(a, b), jnp.minimum(a, b)
    c, d = jnp.maximum(c, d), jnp.minimum(c, d)
    a, c = jnp.maximum(a, c), jnp.minimum(a, c)
    b, d = jnp.maximum(b, d), jnp.minimum(b, d)
    b, c = jnp.maximum(b, c), jnp.minimum(b, c)
    sim_ref[0 * ML:1 * ML] = a
    sim_ref[1 * ML:2 * ML] = b
    sim_ref[2 * ML:3 * ML] = c
    sim_ref[3 * ML:4 * ML] = d

    vs, bs = [], []
    before = jnp.zeros((W,), jnp.float32)
    m = jnp.max(a, axis=0)
    one = jnp.ones((), jnp.bfloat16)
    zero = jnp.zeros((), jnp.bfloat16)
    ones_row = jnp.ones((1, ML), jnp.bfloat16)
    for i in range(K1):
        vs.append(m.astype(jnp.float32) * rq)
        bs.append(before)
        if i < K1 - 1:
            t0 = sim_ref[0 * ML:1 * ML]
            t1 = sim_ref[1 * ML:2 * ML]
            t2 = sim_ref[2 * ML:3 * ML]
            t3 = sim_ref[3 * ML:4 * ML]
            eq = t0 == m[None, :]
            eqb = jnp.where(eq, one, zero)
            cnt = jax.lax.dot_general(
                ones_row, eqb, (((1,), (0,)), ((), ())),
                preferred_element_type=jnp.float32)
            before = before + cnt[0]
            nt0 = jnp.where(eq, t1, t0)
            sim_ref[0 * ML:1 * ML] = nt0
            sim_ref[1 * ML:2 * ML] = jnp.where(eq, t2, t1)
            sim_ref[2 * ML:3 * ML] = jnp.where(eq, t3, t2)
            sim_ref[3 * ML:4 * ML] = jnp.where(eq, -jnp.inf, t3)
            m = jnp.max(nt0, axis=0)
    j_iota = jax.lax.broadcasted_iota(jnp.int32, (K1, W), 0).astype(jnp.float32)
    t1 = jnp.full((K1, W), jnp.inf)
    for v, bc in zip(vs, bs):
        t1 = jnp.minimum(t1, jnp.where(bc[None, :] <= j_iota, v[None, :], jnp.inf))
    t1_ref[...] = t1

    i_iota = jax.lax.broadcasted_iota(jnp.int32, (K1, K2), 1)
    for q in range(QB):
        cur = t1_ref[:, q * HW:(q + 1) * HW]
        v2s, b2s = [], []
        before2 = jnp.zeros((K1,), jnp.int32)
        m2 = jnp.max(cur, axis=1)
        for i in range(K2):
            v2s.append(m2)
            b2s.append(before2)
            if i < K2 - 1:
                eq2 = cur == m2[:, None]
                before2 = before2 + jnp.sum(eq2.astype(jnp.int32), axis=1)
                cur = jnp.where(eq2, -jnp.inf, cur)
                m2 = jnp.max(cur, axis=1)
        out = jnp.full((K1, K2), jnp.inf)
        for v, bc in zip(v2s, b2s):
            out = jnp.minimum(out, jnp.where(bc[:, None] <= i_iota, v[:, None], jnp.inf))
        o_ref[q] = out


def kernel(support, query, task_index, special_list, mode, k, k2):
    s5 = support.reshape(N_CLASS, NS, D, HW).transpose(0, 1, 3, 2)
    s5 = s5.reshape(N_CLASS, M, D)
    s5 = jnp.pad(s5, ((0, 0), (0, MP - M), (0, 0))).astype(jnp.bfloat16)
    q5 = query.reshape(NG, QB, D, HW).transpose(0, 2, 1, 3).reshape(NG, D, W)
    q5 = q5.astype(jnp.bfloat16)

    out = pl.pallas_call(
        _body,
        grid=(N_CLASS, NG),
        in_specs=[
            pl.BlockSpec((1, MP, D), lambda c, g: (c, 0, 0)),
            pl.BlockSpec((1, D, W), lambda c, g: (g, 0, 0)),
        ],
        out_specs=pl.BlockSpec((QB, K1, K2), lambda c, g: (c * NG + g, 0, 0)),
        out_shape=jax.ShapeDtypeStruct((N_CLASS * BQ, K1, K2), jnp.float32),
        scratch_shapes=[pltpu.VMEM((MP, W), jnp.bfloat16),
                        pltpu.VMEM((K1, W), jnp.float32)],
    )(s5, q5)

    zero = (jnp.asarray(k) - K1) + (jnp.asarray(k2) - K2)
    return out.reshape(N_CLASS * BQ, 1, K1, K2) + zero.astype(out.dtype)
